# baseline (speedup 1.0000x reference)
import functools

_CHUNK = 80

import jax
import jax.numpy as jnp
from jax import lax
from jax.experimental import pallas as pl
from jax.experimental.pallas import tpu as pltpu
from jax.experimental.pallas import tpu_sc as plsc


def _prepass_call(x, WT, bias, N, D, R):
    H = D // 2
    nblocks = N // R

    def body(x_ref, wt_ref, b_ref, a_ref, bt_ref, uh_ref):
        xb = x_ref[...]
        a_ref[0] = jnp.dot(xb, wt_ref[0], preferred_element_type=jnp.float32) + b_ref[0:1, :]
        a_ref[1] = jnp.dot(xb, wt_ref[1], preferred_element_type=jnp.float32) + b_ref[1:2, :]
        bt_ref[...] = jnp.dot(xb, wt_ref[2], preferred_element_type=jnp.float32) + b_ref[2:3, :]
        uh_ref[...] = jnp.dot(xb, wt_ref[3], preferred_element_type=jnp.float32) + b_ref[3:4, :]

    return pl.pallas_call(
        body,
        grid=(nblocks,),
        in_specs=[
            pl.BlockSpec((R, D), lambda b: (b, 0)),
            pl.BlockSpec((4, D, D), lambda b: (0, 0, 0)),
            pl.BlockSpec((4, D), lambda b: (0, 0)),
        ],
        out_specs=[
            pl.BlockSpec((2, R, D), lambda b: (0, b, 0)),
            pl.BlockSpec((R, D), lambda b: (b, 0)),
            pl.BlockSpec((R, D), lambda b: (b, 0)),
        ],
        out_shape=[
            jax.ShapeDtypeStruct((2, N, D), jnp.float32),
            jax.ShapeDtypeStruct((N, D), jnp.float32),
            jax.ShapeDtypeStruct((N, D), jnp.float32),
        ],
    )(x, WT, bias)


def _sc_edge_call(A, B, sd3, zeros, N, Np, E, D):
    H = D // 2
    NS = 16
    K = _CHUNK
    Eps = E // NS
    nch = Eps // K
    rps = Np // NS

    mesh = plsc.VectorSubcoreMesh(core_axis_name="c", subcore_axis_name="s")

    @functools.partial(
        pl.kernel,
        out_type=jax.ShapeDtypeStruct((2 * Np, D), jnp.float32),
        mesh=mesh,
        scratch_types=[
            pltpu.VMEM_SHARED((Np, D), jnp.float32),
            [pltpu.VMEM((K, D), jnp.float32) for _ in range(2)],
            [pltpu.VMEM((K, D), jnp.float32) for _ in range(2)],
            [pltpu.VMEM((2, K), jnp.int32) for _ in range(4)],
            [pltpu.SemaphoreType.DMA for _ in range(4)],
            [pltpu.SemaphoreType.DMA for _ in range(2)],
            [pltpu.SemaphoreType.DMA for _ in range(2)],
            [pltpu.SemaphoreType.DMA for _ in range(2)],
        ],
    )
    def sc(a_hbm, b_hbm, sd_hbm, zero_hbm, out_hbm,
           acc, a_bufs, b_bufs, sdidx, qsem, sa, sb, so):
        c = lax.axis_index("c")
        s = lax.axis_index("s")
        row0 = s * rps

        def idx_start(i, q):
            pltpu.async_copy(sd_hbm.at[c, s, i], sdidx[q], qsem[q])

        def idx_wait_and_offset(i, q):
            pltpu.make_async_copy(sd_hbm.at[c, s, i], sdidx[q], qsem[q]).wait()

        def compute(p, c64):
            av, bv = a_bufs[p], b_bufs[p]

            def edge(kk, carry):
                for u in range(4):
                    k = kk * 4 + u
                    for j in range(H // 16):
                        sl = pl.ds(j * 16, 16)
                        slb = pl.ds(c64 + j * 16, 16)
                        sl2 = pl.ds(H + j * 16, 16)
                        e = av[k, sl] + bv[k, slb]
                        sig = 1.0 / (1.0 + jnp.exp(-e))
                        av[k, sl] = sig * av[k, sl2]
                        av[k, sl2] = sig
                return carry

            lax.fori_loop(0, K // 4, edge, 0)

        pltpu.sync_copy(zero_hbm.at[pl.ds(row0, rps)], acc.at[pl.ds(row0, rps)])
        idx_start(0, 0)
        idx_start(1, 1)
        plsc.subcore_barrier()
        idx_wait_and_offset(0, 0)
        pltpu.async_copy(a_hbm.at[sdidx[0].at[0]], a_bufs[0], sa[0])
        pltpu.async_copy(b_hbm.at[sdidx[0].at[1]], b_bufs[0], sb[0])

        def maybe_when(cond, fn):
            if isinstance(cond, bool):
                if cond:
                    fn()
            else:
                pl.when(cond)(fn)

        def body(i, p, q0, q1, q2):
            maybe_when(i + 1 < nch, lambda: idx_wait_and_offset(i + 1, q1))

            def _wait_prev_scatter():
                pltpu.make_async_copy(a_bufs[p ^ 1], acc.at[sdidx[(q0 - 1) % 4].at[1]],
                                      so[p ^ 1]).wait()

            maybe_when(i >= 1, _wait_prev_scatter)

            def _start_next_gather():
                pltpu.async_copy(a_hbm.at[sdidx[q1].at[0]], a_bufs[p ^ 1], sa[p ^ 1])
                pltpu.async_copy(b_hbm.at[sdidx[q1].at[1]], b_bufs[p ^ 1], sb[p ^ 1])

            maybe_when(i + 1 < nch, _start_next_gather)

            pltpu.make_async_copy(a_hbm.at[sdidx[q0].at[0]], a_bufs[p], sa[p]).wait()
            pltpu.make_async_copy(b_hbm.at[sdidx[q0].at[1]], b_bufs[p], sb[p]).wait()

            maybe_when(i + 2 < nch, lambda: idx_start(i + 2, q2))

            @pl.when(c == 0)
            def _():
                compute(p, 0)

            @pl.when(c == 1)
            def _():
                compute(p, H)

            pltpu.async_copy(a_bufs[p], acc.at[sdidx[q0].at[1]], so[p], add=True)

        def quad(g, carry):
            for r in range(4):
                i = g * 4 + r
                body(i, r % 2, r, (r + 1) % 4, (r + 2) % 4)
            return carry

        nquad = (nch // 4) * 4
        lax.fori_loop(0, nch // 4, quad, 0)
        for i in range(nquad, nch):
            body(i, i % 2, i % 4, (i + 1) % 4, (i + 2) % 4)
        pltpu.make_async_copy(a_bufs[1], acc.at[sdidx[(nch - 1) % 4].at[1]], so[1]).wait()
        plsc.subcore_barrier()
        out0 = c * Np
        pltpu.sync_copy(acc.at[pl.ds(row0, rps)],
                        out_hbm.at[pl.ds(out0 + row0, rps)])

    return sc(A, B, sd3, zeros)


def _postpass_call(x, Uh, ACC, gamma, beta, N, Np, D, R):
    H = D // 2
    nblocks = N // R
    off1 = Np // R

    def body(x_ref, uh_ref, a0_ref, a1_ref, g_ref, be_ref, o_ref):
        m = jnp.concatenate([a0_ref[:, :H], a1_ref[:, :H]], axis=1)
        ssum = jnp.concatenate([a0_ref[:, H:], a1_ref[:, H:]], axis=1)
        h = uh_ref[...] + m / (ssum + 1e-6)
        mu = jnp.mean(h, axis=1, keepdims=True)
        d = h - mu
        var = jnp.mean(d * d, axis=1, keepdims=True)
        hn = d * lax.rsqrt(var + 1e-6) * g_ref[0:1, :] + be_ref[0:1, :]
        o_ref[...] = x_ref[...] + jnp.maximum(hn, 0.0)

    return pl.pallas_call(
        body,
        grid=(nblocks,),
        in_specs=[
            pl.BlockSpec((R, D), lambda b: (b, 0)),
            pl.BlockSpec((R, D), lambda b: (b, 0)),
            pl.BlockSpec((R, D), lambda b: (b, 0)),
            pl.BlockSpec((R, D), lambda b: (b + off1, 0)),
            pl.BlockSpec((1, D), lambda b: (0, 0)),
            pl.BlockSpec((1, D), lambda b: (0, 0)),
        ],
        out_specs=pl.BlockSpec((R, D), lambda b: (b, 0)),
        out_shape=jax.ShapeDtypeStruct((N, D), jnp.float32),
    )(x, Uh, ACC, ACC, gamma, beta)


def kernel(x, edge_index, W_w, W_b, U_w, U_b, V_w, V_b, attn_l, attn_r, gamma, beta):
    N, D = x.shape
    E = edge_index.shape[1]
    R = 400

    H = D // 2
    al, ar = attn_l[0], attn_r[0]
    Wt, Ut, Vt = W_w.T, U_w.T, V_w.T
    M0 = jnp.concatenate([Wt[:, :H] * al[:H], Vt[:, :H]], axis=1)
    c0 = jnp.concatenate([W_b[:H] * al[:H], V_b[:H]])
    M1 = jnp.concatenate([Wt[:, H:] * al[H:], Vt[:, H:]], axis=1)
    c1 = jnp.concatenate([W_b[H:] * al[H:], V_b[H:]])
    WT = jnp.stack([M0, M1, Wt * ar, Ut])
    bias = jnp.stack([c0, c1, W_b * ar, U_b])

    A, B, Uh = _prepass_call(x, WT, bias, N, D, R)
    A = A.reshape(2 * N, D)

    NS, K = 16, _CHUNK
    nch = E // (NS * K)
    sd3 = jnp.transpose(edge_index.reshape(2, NS, nch, K), (1, 2, 0, 3))
    sd3 = jnp.stack([sd3, sd3.at[:, :, 0, :].add(N)])
    Np = 10240
    zeros = jnp.zeros((Np, D), jnp.float32)

    ACC = _sc_edge_call(A, B, sd3, zeros, N, Np, E, D)

    return _postpass_call(x, Uh, ACC, gamma.reshape(1, D), beta.reshape(1, D),
                          N, Np, D, 80)


# speedup vs baseline: 1.2285x; 1.2285x over previous
import functools

_CHUNK = 80

import jax
import jax.numpy as jnp
from jax import lax
from jax.experimental import pallas as pl
from jax.experimental.pallas import tpu as pltpu
from jax.experimental.pallas import tpu_sc as plsc


def _prepass_call(x, WT, bias, N, D, R):
    H = D // 2
    nblocks = N // R

    def body(x_ref, wt_ref, b_ref, a_ref, bt_ref, uh_ref):
        xb = x_ref[...]
        a_ref[0] = jnp.dot(xb, wt_ref[0], preferred_element_type=jnp.float32) + b_ref[0:1, :]
        a_ref[1] = jnp.dot(xb, wt_ref[1], preferred_element_type=jnp.float32) + b_ref[1:2, :]
        bt_ref[...] = jnp.dot(xb, wt_ref[2], preferred_element_type=jnp.float32) + b_ref[2:3, :]
        uh_ref[...] = jnp.dot(xb, wt_ref[3], preferred_element_type=jnp.float32) + b_ref[3:4, :]

    return pl.pallas_call(
        body,
        grid=(nblocks,),
        in_specs=[
            pl.BlockSpec((R, D), lambda b: (b, 0)),
            pl.BlockSpec((4, D, D), lambda b: (0, 0, 0)),
            pl.BlockSpec((4, D), lambda b: (0, 0)),
        ],
        out_specs=[
            pl.BlockSpec((2, R, D), lambda b: (0, b, 0)),
            pl.BlockSpec((R, D), lambda b: (b, 0)),
            pl.BlockSpec((R, D), lambda b: (b, 0)),
        ],
        out_shape=[
            jax.ShapeDtypeStruct((2, N, D), jnp.float32),
            jax.ShapeDtypeStruct((N, D), jnp.float32),
            jax.ShapeDtypeStruct((N, D), jnp.float32),
        ],
    )(x, WT, bias)


def _sc_edge_call(A, B, sd3, zeros, N, Np, E, D):
    H = D // 2
    NS = 16
    K = _CHUNK
    Eps = E // NS
    nch = Eps // K
    rps = Np // NS

    mesh = plsc.VectorSubcoreMesh(core_axis_name="c", subcore_axis_name="s")

    @functools.partial(
        pl.kernel,
        out_type=jax.ShapeDtypeStruct((2 * Np, D), jnp.float32),
        mesh=mesh,
        scratch_types=[
            pltpu.VMEM_SHARED((Np, D), jnp.float32),
            [pltpu.VMEM((K, D), jnp.float32) for _ in range(2)],
            [pltpu.VMEM((K, D), jnp.float32) for _ in range(2)],
            [pltpu.VMEM((2, K), jnp.int32) for _ in range(4)],
            [pltpu.SemaphoreType.DMA for _ in range(4)],
            [pltpu.SemaphoreType.DMA for _ in range(2)],
            [pltpu.SemaphoreType.DMA for _ in range(2)],
            [pltpu.SemaphoreType.DMA for _ in range(2)],
        ],
    )
    def sc(a_hbm, b_hbm, sd_hbm, zero_hbm, out_hbm,
           acc, a_bufs, b_bufs, sdidx, qsem, sa, sb, so):
        c = lax.axis_index("c")
        s = lax.axis_index("s")
        row0 = s * rps
        node0 = c * N

        def idx_start(i, q):
            pltpu.async_copy(sd_hbm.at[s, i], sdidx[q], qsem[q])

        def idx_wait_and_offset(i, q):
            pltpu.make_async_copy(sd_hbm.at[s, i], sdidx[q], qsem[q]).wait()
            for j in range(K // 16):
                sl = pl.ds(j * 16, 16)
                sdidx[q][0, sl] = sdidx[q][0, sl] + node0

        def compute(p, c64):
            av, bv = a_bufs[p], b_bufs[p]

            @functools.partial(plsc.parallel_loop, 0, K, unroll=4)
            def _(k):
                for j in range(H // 16):
                    sl = pl.ds(j * 16, 16)
                    slb = pl.ds(c64 + j * 16, 16)
                    sl2 = pl.ds(H + j * 16, 16)
                    e = av[k, sl] + bv[k, slb]
                    sig = 1.0 / (1.0 + jnp.exp(-e))
                    av[k, sl] = sig * av[k, sl2]
                    av[k, sl2] = sig

        pltpu.sync_copy(zero_hbm.at[pl.ds(row0, rps)], acc.at[pl.ds(row0, rps)])
        idx_start(0, 0)
        idx_start(1, 1)
        plsc.subcore_barrier()
        idx_wait_and_offset(0, 0)
        pltpu.async_copy(a_hbm.at[sdidx[0].at[0]], a_bufs[0], sa[0])
        pltpu.async_copy(b_hbm.at[sdidx[0].at[1]], b_bufs[0], sb[0])

        def maybe_when(cond, fn):
            if isinstance(cond, bool):
                if cond:
                    fn()
            else:
                pl.when(cond)(fn)

        def body(i, p, q0, q1, q2):
            maybe_when(i + 1 < nch, lambda: idx_wait_and_offset(i + 1, q1))

            def _wait_prev_scatter():
                pltpu.make_async_copy(a_bufs[p ^ 1], acc.at[sdidx[(q0 - 1) % 4].at[1]],
                                      so[p ^ 1]).wait()

            maybe_when(i >= 1, _wait_prev_scatter)

            def _start_next_gather():
                pltpu.async_copy(a_hbm.at[sdidx[q1].at[0]], a_bufs[p ^ 1], sa[p ^ 1])
                pltpu.async_copy(b_hbm.at[sdidx[q1].at[1]], b_bufs[p ^ 1], sb[p ^ 1])

            maybe_when(i + 1 < nch, _start_next_gather)

            pltpu.make_async_copy(a_hbm.at[sdidx[q0].at[0]], a_bufs[p], sa[p]).wait()
            pltpu.make_async_copy(b_hbm.at[sdidx[q0].at[1]], b_bufs[p], sb[p]).wait()

            maybe_when(i + 2 < nch, lambda: idx_start(i + 2, q2))

            @pl.when(c == 0)
            def _():
                compute(p, 0)

            @pl.when(c == 1)
            def _():
                compute(p, H)

            pltpu.async_copy(a_bufs[p], acc.at[sdidx[q0].at[1]], so[p], add=True)

        def quad(g, carry):
            for r in range(4):
                i = g * 4 + r
                body(i, r % 2, r, (r + 1) % 4, (r + 2) % 4)
            return carry

        nquad = (nch // 4) * 4
        lax.fori_loop(0, nch // 4, quad, 0)
        for i in range(nquad, nch):
            body(i, i % 2, i % 4, (i + 1) % 4, (i + 2) % 4)
        pltpu.make_async_copy(a_bufs[1], acc.at[sdidx[(nch - 1) % 4].at[1]], so[1]).wait()
        plsc.subcore_barrier()
        out0 = c * Np
        pltpu.sync_copy(acc.at[pl.ds(row0, rps)],
                        out_hbm.at[pl.ds(out0 + row0, rps)])

    return sc(A, B, sd3, zeros)


def _postpass_call(x, Uh, ACC, gamma, beta, N, Np, D, R):
    H = D // 2
    nblocks = N // R
    off1 = Np // R

    def body(x_ref, uh_ref, a0_ref, a1_ref, g_ref, be_ref, o_ref):
        m = jnp.concatenate([a0_ref[:, :H], a1_ref[:, :H]], axis=1)
        ssum = jnp.concatenate([a0_ref[:, H:], a1_ref[:, H:]], axis=1)
        h = uh_ref[...] + m / (ssum + 1e-6)
        mu = jnp.mean(h, axis=1, keepdims=True)
        d = h - mu
        var = jnp.mean(d * d, axis=1, keepdims=True)
        hn = d * lax.rsqrt(var + 1e-6) * g_ref[0:1, :] + be_ref[0:1, :]
        o_ref[...] = x_ref[...] + jnp.maximum(hn, 0.0)

    return pl.pallas_call(
        body,
        grid=(nblocks,),
        in_specs=[
            pl.BlockSpec((R, D), lambda b: (b, 0)),
            pl.BlockSpec((R, D), lambda b: (b, 0)),
            pl.BlockSpec((R, D), lambda b: (b, 0)),
            pl.BlockSpec((R, D), lambda b: (b + off1, 0)),
            pl.BlockSpec((1, D), lambda b: (0, 0)),
            pl.BlockSpec((1, D), lambda b: (0, 0)),
        ],
        out_specs=pl.BlockSpec((R, D), lambda b: (b, 0)),
        out_shape=jax.ShapeDtypeStruct((N, D), jnp.float32),
    )(x, Uh, ACC, ACC, gamma, beta)


def kernel(x, edge_index, W_w, W_b, U_w, U_b, V_w, V_b, attn_l, attn_r, gamma, beta):
    N, D = x.shape
    E = edge_index.shape[1]
    R = 400

    H = D // 2
    al, ar = attn_l[0], attn_r[0]
    Wt, Ut, Vt = W_w.T, U_w.T, V_w.T
    M0 = jnp.concatenate([Wt[:, :H] * al[:H], Vt[:, :H]], axis=1)
    c0 = jnp.concatenate([W_b[:H] * al[:H], V_b[:H]])
    M1 = jnp.concatenate([Wt[:, H:] * al[H:], Vt[:, H:]], axis=1)
    c1 = jnp.concatenate([W_b[H:] * al[H:], V_b[H:]])
    WT = jnp.stack([M0, M1, Wt * ar, Ut])
    bias = jnp.stack([c0, c1, W_b * ar, U_b])

    A, B, Uh = _prepass_call(x, WT, bias, N, D, R)
    A = A.reshape(2 * N, D)

    NS, K = 16, _CHUNK
    nch = E // (NS * K)
    sd3 = jnp.transpose(edge_index.reshape(2, NS, nch, K), (1, 2, 0, 3))
    Np = 10240
    zeros = jnp.zeros((Np, D), jnp.float32)

    ACC = _sc_edge_call(A, B, sd3, zeros, N, Np, E, D)

    return _postpass_call(x, Uh, ACC, gamma.reshape(1, D), beta.reshape(1, D),
                          N, Np, D, 80)
